# Initial kernel scaffold; baseline (speedup 1.0000x reference)
import jax
import jax.numpy as jnp
from jax.experimental import pallas as pl


def kernel(x, edge_index, batch, n_test, n_pred, W1, b1, W2, b2, Wf1, bf1, Wf2, bf2, Wf3, bf3, Wo, bo):
    raise NotImplementedError("write your pallas kernel here")


# speedup vs baseline: 17.9781x; 17.9781x over previous
import functools

import jax
import jax.numpy as jnp
from jax import lax
from jax.experimental import pallas as pl
from jax.experimental.pallas import tpu as pltpu
from jax.experimental.pallas import tpu_sc as plsc

N = 10000
E = 320000
NG = 64
F_IN = 128
DW = 56
ONES_COL = 50

NC = 2
NS = 16
NW = NC * NS
CHUNK = 128
NCH = 80
EPT = NCH * CHUNK
EPAD = NW * EPT
PAD_NODE = 10000
AROWS = 10016
RPT = AROWS // NS
NPAD = 10240
DRPT = NPAD // NS

_MESH = plsc.VectorSubcoreMesh(core_axis_name="c", subcore_axis_name="s")
_F32 = jnp.float32
_CP = pltpu.CompilerParams(use_tc_tiling_on_sc=False)


@functools.partial(
    pl.kernel,
    mesh=_MESH,
    out_type=jax.ShapeDtypeStruct((NC * NPAD,), _F32),
    compiler_params=_CP,
    scratch_types=[
        pltpu.VMEM((NCH, CHUNK), jnp.int32),
        pltpu.VMEM((CHUNK,), _F32),
        pltpu.VMEM((DRPT,), _F32),
        pltpu.VMEM_SHARED((NPAD,), _F32),
    ],
)
def _deg_kernel(dst_hbm, ones_hbm, out_hbm, idx_v, ones_v, bounce_v, acc_sh):
    cid = lax.axis_index("c")
    sid = lax.axis_index("s")
    wid = cid * NS + sid
    pltpu.sync_copy(dst_hbm.at[wid], idx_v)
    pltpu.sync_copy(ones_hbm, ones_v)

    def zero_body(i, c):
        bounce_v[pl.ds(i * 16, 16)] = jnp.zeros((16,), _F32)
        return c

    lax.fori_loop(0, DRPT // 16, zero_body, 0)
    pltpu.sync_copy(bounce_v, acc_sh.at[pl.ds(sid * DRPT, DRPT)])
    plsc.subcore_barrier()

    def body(j, c):
        pltpu.sync_copy(ones_v, acc_sh.at[idx_v.at[j]], add=True)
        return c

    lax.fori_loop(0, NCH, body, 0)
    plsc.subcore_barrier()
    pltpu.sync_copy(acc_sh.at[pl.ds(sid * DRPT, DRPT)], bounce_v)
    pltpu.sync_copy(bounce_v, out_hbm.at[pl.ds(cid * NPAD + sid * DRPT, DRPT)])


@functools.partial(
    pl.kernel,
    mesh=_MESH,
    out_type=jax.ShapeDtypeStruct((NC, AROWS, DW), _F32),
    compiler_params=_CP,
    scratch_types=[
        pltpu.VMEM((NCH, CHUNK), jnp.int32),
        pltpu.VMEM((NCH, CHUNK), jnp.int32),
        pltpu.VMEM((CHUNK, DW), _F32),
        pltpu.VMEM((CHUNK, DW), _F32),
        pltpu.VMEM((RPT, DW), _F32),
        pltpu.VMEM_SHARED((AROWS, DW), _F32),
        pltpu.SemaphoreType.DMA,
        pltpu.SemaphoreType.DMA,
    ],
)
def _prop_kernel(src_hbm, dst_hbm, feat_hbm, out_hbm, src_v, dst_v, rows_a,
                 rows_b, bounce_v, acc_sh, sem_a, sem_b):
    cid = lax.axis_index("c")
    sid = lax.axis_index("s")
    wid = cid * NS + sid
    pltpu.sync_copy(src_hbm.at[wid], src_v)
    pltpu.sync_copy(dst_hbm.at[wid], dst_v)

    def zero_body(i, c):
        for off in (0, 16, 32, 40):
            bounce_v[i, pl.ds(off, 16)] = jnp.zeros((16,), _F32)
        return c

    lax.fori_loop(0, RPT, zero_body, 0)
    pltpu.sync_copy(bounce_v, acc_sh.at[pl.ds(sid * RPT, RPT)])
    plsc.subcore_barrier()

    pltpu.async_copy(feat_hbm.at[src_v.at[0]], rows_a, sem_a)

    def body(h, c):
        j = h * 2
        pltpu.make_async_copy(feat_hbm.at[src_v.at[j]], rows_a, sem_a).wait()
        pltpu.async_copy(feat_hbm.at[src_v.at[j + 1]], rows_b, sem_b)
        pltpu.sync_copy(rows_a, acc_sh.at[dst_v.at[j]], add=True)
        pltpu.make_async_copy(feat_hbm.at[src_v.at[j + 1]], rows_b,
                              sem_b).wait()

        @pl.when(h + 1 < NCH // 2)
        def _():
            pltpu.async_copy(feat_hbm.at[src_v.at[j + 2]], rows_a, sem_a)

        pltpu.sync_copy(rows_b, acc_sh.at[dst_v.at[j + 1]], add=True)
        return c

    lax.fori_loop(0, NCH // 2, body, 0)
    plsc.subcore_barrier()
    pltpu.sync_copy(acc_sh.at[pl.ds(sid * RPT, RPT)], bounce_v)
    pltpu.sync_copy(bounce_v, out_hbm.at[cid, pl.ds(sid * RPT, RPT)])


def _k2_body(x_ref, w1_ref, w2_ref, degp_ref, yt_ref, dinv_ref):
    w12 = jnp.dot(w1_ref[...], w2_ref[...], preferred_element_type=_F32)
    w12p = jnp.concatenate(
        [w12, jnp.zeros((F_IN, DW - w12.shape[1]), _F32)], axis=1)
    y = jnp.dot(x_ref[...], w12p, preferred_element_type=_F32)
    degp = degp_ref[...]
    deg = 1.0 + degp[:N] + degp[NPAD:NPAD + N]
    dinv = lax.rsqrt(deg).reshape(N, 1)
    cols = lax.broadcasted_iota(jnp.int32, (N, DW), 1)
    ones_chan = jnp.where(cols == ONES_COL, 1.0, 0.0)
    yt = dinv * (y + ones_chan)
    yt_ref[...] = jnp.concatenate(
        [yt, jnp.zeros((AROWS - N, DW), _F32)], axis=0)
    dinv_ref[...] = dinv


_k2_call = pl.pallas_call(
    _k2_body,
    out_shape=[
        jax.ShapeDtypeStruct((AROWS, DW), _F32),
        jax.ShapeDtypeStruct((N, 1), _F32),
    ],
)


def _k4_body(p_ref, yt_ref, dinv_ref, z_ref):
    dinv = dinv_ref[...]
    z = (dinv * dinv) * (p_ref[0, :N, :] + p_ref[1, :N, :] + yt_ref[:N, :])
    z_ref[...] = jnp.concatenate([z, jnp.zeros((AROWS - N, DW), _F32)], axis=0)


_k4_call = pl.pallas_call(
    _k4_body,
    out_shape=jax.ShapeDtypeStruct((AROWS, DW), _F32),
)


def _k6_body(q_ref, z1t_ref, dinv_ref, batch_ref, w2_ref, b1_ref, b2_ref,
             wf1_ref, bf1_ref, wf2_ref, bf2_ref, wf3_ref, bf3_ref, wo_ref,
             bo_ref, out_ref):
    dinv = dinv_ref[...]
    z1t = z1t_ref[:N, :]
    z2 = dinv * (q_ref[0, :N, :] + q_ref[1, :N, :] + z1t)
    r1 = z1t[:, ONES_COL:ONES_COL + 1] / dinv
    cols = lax.broadcasted_iota(jnp.int32, (N, DW), 1)
    m = jnp.where(cols == 51, r1, jnp.where(cols == 52, 1.0, z2))
    batch = batch_ref[...].reshape(N, 1)
    seg = lax.broadcasted_iota(jnp.int32, (N, NG), 1)
    onehot = (batch == seg).astype(_F32)
    g = lax.dot_general(onehot, m, (((0,), (0,)), ((), ())),
                        preferred_element_type=_F32)
    b1w2 = jnp.dot(b1_ref[...].reshape(1, -1), w2_ref[...],
                   preferred_element_type=_F32)
    pre = g[:, :50] + g[:, 51:52] * b1w2 \
        + g[:, 52:53] * b2_ref[...].reshape(1, -1)
    h = jnp.dot(pre, wf1_ref[...], preferred_element_type=_F32)
    h = h + bf1_ref[...].reshape(1, -1)
    h = jnp.dot(h, wf2_ref[...], preferred_element_type=_F32)
    h = h + bf2_ref[...].reshape(1, -1)
    h = jnp.dot(h, wf3_ref[...], preferred_element_type=_F32)
    h = h + bf3_ref[...].reshape(1, -1)
    out_ref[...] = jnp.dot(h, wo_ref[...], preferred_element_type=_F32) \
        + bo_ref[...].reshape(1, -1)


_k6_call = pl.pallas_call(
    _k6_body,
    out_shape=jax.ShapeDtypeStruct((NG, 1), _F32),
)


def kernel(x, edge_index, batch, n_test, n_pred, W1, b1, W2, b2, Wf1, bf1,
           Wf2, bf2, Wf3, bf3, Wo, bo):
    del n_test, n_pred
    pad = jnp.full((EPAD - E,), PAD_NODE, jnp.int32)
    src3 = jnp.concatenate([edge_index[0], pad]).reshape(NW, NCH, CHUNK)
    dst3 = jnp.concatenate([edge_index[1], pad]).reshape(NW, NCH, CHUNK)
    ones_chunk = jnp.ones((CHUNK,), _F32)
    degp = _deg_kernel(dst3, ones_chunk)
    yt, dinv = _k2_call(x, W1, W2, degp)
    p = _prop_kernel(src3, dst3, yt)
    z1t = _k4_call(p, yt, dinv)
    q = _prop_kernel(src3, dst3, z1t)
    return _k6_call(q, z1t, dinv, batch, W2, b1, b2, Wf1, bf1, Wf2, bf2,
                    Wf3, bf3, Wo, bo)
